# Initial kernel scaffold; baseline (speedup 1.0000x reference)
import jax
import jax.numpy as jnp
from jax.experimental import pallas as pl


def kernel(x, edge_index, edge_attr, batch, atom_tables, bond_tables, eps, mlp_W1, mlp_b1, mlp_bn_g, mlp_bn_b, mlp_W2, mlp_b2, bn_g, bn_b, lin_W, lin_b):
    raise NotImplementedError("write your pallas kernel here")


# speedup vs baseline: 4.2005x; 4.2005x over previous
import functools

import jax
import jax.numpy as jnp
from jax import lax
from jax.experimental import pallas as pl
from jax.experimental.pallas import tpu as pltpu
from jax.experimental.pallas import tpu_sc as plsc

N = 50000
E = 800000
EMB = 64
HALF = 32
L = 5
G = 128
BN_EPS = 1e-5

NC = 2
NS = 16
CH = 128

NP = 51200
NCHUNK_N = NP // (NS * CH)
EW = 50304
EPAD = EW * NS
NCHUNK_E = EW // CH

RB = 2048
NBLK = NP // RB

@functools.lru_cache(maxsize=None)
def _sc_mesh():
    return plsc.VectorSubcoreMesh(
        core_axis_name="c", subcore_axis_name="s",
        num_cores=NC, num_subcores=NS)


def _fill_zero(buf, rows):
    z16 = jnp.zeros((16,), jnp.float32)

    def body(i, _):
        buf[i, pl.ds(0, 16)] = z16
        buf[i, pl.ds(16, 16)] = z16
        return 0
    lax.fori_loop(0, rows, body, 0, unroll=8)


def _atom_body(xt_hbm, atab_hbm, h_hbm, idxs, bufs, hout, sem):
    c = lax.axis_index("c")
    s = lax.axis_index("s")
    cbase = c * (9 * 128)

    def chunk(t, _):
        base = (s * NCHUNK_N + t) * CH
        for i in range(9):
            pltpu.sync_copy(xt_hbm.at[pl.ds(i * NP + base, CH)], idxs.at[i])

        def offs(j, _):
            for i in range(9):
                idxs[i, pl.ds(j * 16, 16)] = (
                    idxs[i, pl.ds(j * 16, 16)] + cbase + i * 128)
            return 0
        lax.fori_loop(0, CH // 16, offs, 0)

        cps = [
            pltpu.async_copy(atab_hbm.at[idxs.at[i]],
                             bufs.at[pl.ds(i * CH, CH)], sem)
            for i in range(9)
        ]
        for cp in cps:
            cp.wait()

        def rowsum(r, _):
            for g in range(2):
                sl = pl.ds(g * 16, 16)
                v = bufs[r, sl]
                for i in range(1, 9):
                    v = v + bufs[i * CH + r, sl]
                hout[r, sl] = v
            return 0
        lax.fori_loop(0, CH, rowsum, 0, unroll=4)

        pltpu.sync_copy(hout, h_hbm.at[pl.ds(c * NP + base, CH)])
        return 0

    lax.fori_loop(0, NCHUNK_N, chunk, 0)


@functools.lru_cache(maxsize=None)
def _atom_kernel_fn():
    return pl.kernel(
        _atom_body,
        out_type=jax.ShapeDtypeStruct((NC * NP, HALF), jnp.float32),
        mesh=_sc_mesh(),
        scratch_types=[
            pltpu.VMEM((9, CH), jnp.int32),
            pltpu.VMEM((9 * CH, HALF), jnp.float32),
            pltpu.VMEM((CH, HALF), jnp.float32),
            pltpu.SemaphoreType.DMA,
        ],
        compiler_params=pltpu.CompilerParams(use_tc_tiling_on_sc=False),
    )


def _atom_kernel(xt, atab):
    return _atom_kernel_fn()(xt, atab)


def _msg_body(h_hbm, src_hbm, dst_hbm, cidx_hbm, eetab_hbm, agg_hbm,
              src_v, dst_v, cidx_v, hbuf, eebuf, acc, sem_h, sem_e):
    c = lax.axis_index("c")
    s = lax.axis_index("s")

    _fill_zero(hbuf, CH)

    def zchunk(t, _):
        r = (s * NCHUNK_N + t) * CH
        pltpu.sync_copy(hbuf, acc.at[pl.ds(r, CH)])
        return 0
    lax.fori_loop(0, NCHUNK_N, zchunk, 0)
    plsc.subcore_barrier()

    hoff = c * NP
    eoff = c * 4096

    def chunk(k, _):
        base = s * EW + k * CH
        pltpu.sync_copy(src_hbm.at[pl.ds(base, CH)], src_v)
        pltpu.sync_copy(cidx_hbm.at[pl.ds(base, CH)], cidx_v)
        pltpu.sync_copy(dst_hbm.at[pl.ds(base, CH)], dst_v)

        def offs(j, _):
            sl = pl.ds(j * 16, 16)
            src_v[sl] = src_v[sl] + hoff
            cidx_v[sl] = cidx_v[sl] + eoff
            return 0
        lax.fori_loop(0, CH // 16, offs, 0)

        cp1 = pltpu.async_copy(h_hbm.at[src_v], hbuf, sem_h)
        cp2 = pltpu.async_copy(eetab_hbm.at[cidx_v], eebuf, sem_e)
        cp1.wait()
        cp2.wait()

        def relu_add(r, _):
            for g in range(2):
                sl = pl.ds(g * 16, 16)
                hbuf[r, sl] = jnp.maximum(hbuf[r, sl] + eebuf[r, sl], 0.0)
            return 0
        lax.fori_loop(0, CH, relu_add, 0, unroll=8)

        pltpu.sync_copy(hbuf, acc.at[dst_v], add=True)
        return 0

    lax.fori_loop(0, NCHUNK_E, chunk, 0)
    plsc.subcore_barrier()

    def flush(t, _):
        r = (s * NCHUNK_N + t) * CH
        pltpu.sync_copy(acc.at[pl.ds(r, CH)], agg_hbm.at[pl.ds(c * NP + r, CH)])
        return 0
    lax.fori_loop(0, NCHUNK_N, flush, 0)


@functools.lru_cache(maxsize=None)
def _msg_kernel_fn():
    return pl.kernel(
        _msg_body,
        out_type=jax.ShapeDtypeStruct((NC * NP, HALF), jnp.float32),
        mesh=_sc_mesh(),
        scratch_types=[
            pltpu.VMEM((CH,), jnp.int32),
            pltpu.VMEM((CH,), jnp.int32),
            pltpu.VMEM((CH,), jnp.int32),
            pltpu.VMEM((CH, HALF), jnp.float32),
            pltpu.VMEM((CH, HALF), jnp.float32),
            pltpu.VMEM_SHARED((NP, HALF), jnp.float32),
            pltpu.SemaphoreType.DMA,
            pltpu.SemaphoreType.DMA,
        ],
        compiler_params=pltpu.CompilerParams(use_tc_tiling_on_sc=False),
    )


def _msg_kernel(h, src, dst, cidx, eetab):
    return _msg_kernel_fn()(h, src, dst, cidx, eetab)


def _bond_combine_body(t_ref, out_ref):
    t0 = t_ref[0, 0]
    t1 = t_ref[0, 1]
    t2 = t_ref[0, 2]
    v = (t0[:, None, None, :] + t1[None, :, None, :] + t2[None, None, :, :])
    out_ref[0] = v.reshape(4096, EMB)


def _build_bond_tables(bond_tables):
    return pl.pallas_call(
        _bond_combine_body,
        grid=(L,),
        in_specs=[pl.BlockSpec((1, 3, 16, EMB), lambda l: (l, 0, 0, 0))],
        out_specs=pl.BlockSpec((1, 4096, EMB), lambda l: (l, 0, 0)),
        out_shape=jax.ShapeDtypeStruct((L, 4096, EMB), jnp.float32),
    )(bond_tables)


def _passA_body(h0_ref, h1_ref, a0_ref, a1_ref, w1a_ref, w1b_ref, b1_ref,
                c_ref, y_ref, sums_ref):
    i = pl.program_id(0)
    cm = c_ref[0, 0]
    bf = jnp.bfloat16
    z0 = (cm * h0_ref[...] + a0_ref[...]).astype(bf)
    z1 = (cm * h1_ref[...] + a1_ref[...]).astype(bf)
    y = jnp.dot(z0, w1a_ref[...].astype(bf), preferred_element_type=jnp.float32)
    y = y + jnp.dot(z1, w1b_ref[...].astype(bf),
                    preferred_element_type=jnp.float32)
    y = y + b1_ref[...]
    y_ref[...] = y
    rows = i * RB + lax.broadcasted_iota(jnp.int32, (RB, 1), 0)
    ym = jnp.where(rows < N, y, 0.0)
    part = jnp.concatenate(
        [jnp.sum(ym, axis=0, keepdims=True),
         jnp.sum(ym * ym, axis=0, keepdims=True)], axis=0)

    @pl.when(i == 0)
    def _():
        sums_ref[...] = part

    @pl.when(i > 0)
    def _():
        sums_ref[...] = sums_ref[...] + part


def _passA(h0, h1, a0, a1, w1a, w1b, b1, cmul):
    return pl.pallas_call(
        _passA_body,
        grid=(NBLK,),
        in_specs=[
            pl.BlockSpec((RB, HALF), lambda i: (i, 0)),
            pl.BlockSpec((RB, HALF), lambda i: (i, 0)),
            pl.BlockSpec((RB, HALF), lambda i: (i, 0)),
            pl.BlockSpec((RB, HALF), lambda i: (i, 0)),
            pl.BlockSpec((HALF, EMB), lambda i: (0, 0)),
            pl.BlockSpec((HALF, EMB), lambda i: (0, 0)),
            pl.BlockSpec((1, EMB), lambda i: (0, 0)),
            pl.BlockSpec((1, 1), lambda i: (0, 0)),
        ],
        out_specs=[
            pl.BlockSpec((RB, EMB), lambda i: (i, 0)),
            pl.BlockSpec((2, EMB), lambda i: (0, 0)),
        ],
        out_shape=[
            jax.ShapeDtypeStruct((NP, EMB), jnp.float32),
            jax.ShapeDtypeStruct((2, EMB), jnp.float32),
        ],
    )(h0, h1, a0, a1, w1a, w1b, b1, cmul)


def _passB_body(y_ref, sums_ref, g_ref, b_ref, w2_ref, b2_ref, u_ref,
                sums2_ref):
    i = pl.program_id(0)
    mu = sums_ref[0:1, :] * (1.0 / N)
    var = sums_ref[1:2, :] * (1.0 / N) - mu * mu
    sc = g_ref[...] * lax.rsqrt(var + BN_EPS)
    t = b_ref[...] - mu * sc
    r = jnp.maximum(y_ref[...] * sc + t, 0.0)
    u = jnp.dot(r.astype(jnp.bfloat16), w2_ref[...].astype(jnp.bfloat16),
                preferred_element_type=jnp.float32)
    u = u + b2_ref[...]
    u_ref[...] = u
    rows = i * RB + lax.broadcasted_iota(jnp.int32, (RB, 1), 0)
    um = jnp.where(rows < N, u, 0.0)
    part = jnp.concatenate(
        [jnp.sum(um, axis=0, keepdims=True),
         jnp.sum(um * um, axis=0, keepdims=True)], axis=0)

    @pl.when(i == 0)
    def _():
        sums2_ref[...] = part

    @pl.when(i > 0)
    def _():
        sums2_ref[...] = sums2_ref[...] + part


def _passB(y, sums, g, b, w2, b2):
    return pl.pallas_call(
        _passB_body,
        grid=(NBLK,),
        in_specs=[
            pl.BlockSpec((RB, EMB), lambda i: (i, 0)),
            pl.BlockSpec((2, EMB), lambda i: (0, 0)),
            pl.BlockSpec((1, EMB), lambda i: (0, 0)),
            pl.BlockSpec((1, EMB), lambda i: (0, 0)),
            pl.BlockSpec((EMB, EMB), lambda i: (0, 0)),
            pl.BlockSpec((1, EMB), lambda i: (0, 0)),
        ],
        out_specs=[
            pl.BlockSpec((RB, EMB), lambda i: (i, 0)),
            pl.BlockSpec((2, EMB), lambda i: (0, 0)),
        ],
        out_shape=[
            jax.ShapeDtypeStruct((NP, EMB), jnp.float32),
            jax.ShapeDtypeStruct((2, EMB), jnp.float32),
        ],
    )(y, sums, g, b, w2, b2)


def _passC_body(relu, u_ref, sums_ref, g_ref, b_ref, h_ref):
    c = pl.program_id(0)
    mu = sums_ref[0:1, :] * (1.0 / N)
    var = sums_ref[1:2, :] * (1.0 / N) - mu * mu
    sc = g_ref[...] * lax.rsqrt(var + BN_EPS)
    t = b_ref[...] - mu * sc
    h = u_ref[...] * sc + t
    if relu:
        h = jnp.maximum(h, 0.0)
    h_ref[0] = jnp.where(c == 0, h[:, :HALF], h[:, HALF:])


def _passC(u, sums, g, b, relu):
    return pl.pallas_call(
        functools.partial(_passC_body, relu),
        grid=(NC, NBLK),
        in_specs=[
            pl.BlockSpec((RB, EMB), lambda c, i: (i, 0)),
            pl.BlockSpec((2, EMB), lambda c, i: (0, 0)),
            pl.BlockSpec((1, EMB), lambda c, i: (0, 0)),
            pl.BlockSpec((1, EMB), lambda c, i: (0, 0)),
        ],
        out_specs=pl.BlockSpec((1, RB, HALF), lambda c, i: (c, i, 0)),
        out_shape=jax.ShapeDtypeStruct((NC, NP, HALF), jnp.float32),
    )(u, sums, g, b)


def _pool_body(u_ref, sums_ref, g_ref, b_ref, batch_ref, lw_ref, lb_ref,
               out_ref, acc):
    i = pl.program_id(0)
    mu = sums_ref[0:1, :] * (1.0 / N)
    var = sums_ref[1:2, :] * (1.0 / N) - mu * mu
    sc = g_ref[...] * lax.rsqrt(var + BN_EPS)
    t = b_ref[...] - mu * sc
    h = u_ref[...] * sc + t
    bb = batch_ref[0]
    gid = lax.broadcasted_iota(jnp.int32, (G, RB), 0)
    rows = i * RB + lax.broadcasted_iota(jnp.int32, (G, RB), 1)
    oh = jnp.where((gid == bb) & (rows < N), 1.0, 0.0)
    part = jax.lax.dot_general(oh, h, (((1,), (0,)), ((), ())),
                               preferred_element_type=jnp.float32,
                precision=lax.Precision.HIGHEST)

    @pl.when(i == 0)
    def _():
        acc[...] = part

    @pl.when(i > 0)
    def _():
        acc[...] = acc[...] + part

    out_ref[...] = (
        jnp.dot(acc[...].astype(jnp.bfloat16),
                lw_ref[...].astype(jnp.bfloat16),
                preferred_element_type=jnp.float32)
        + lb_ref[...])


def _pool(u, sums3, g, b, batch3, lin_W, lin_b):
    return pl.pallas_call(
        _pool_body,
        grid=(NBLK,),
        in_specs=[
            pl.BlockSpec((RB, EMB), lambda i: (i, 0)),
            pl.BlockSpec((2, EMB), lambda i: (0, 0)),
            pl.BlockSpec((1, EMB), lambda i: (0, 0)),
            pl.BlockSpec((1, EMB), lambda i: (0, 0)),
            pl.BlockSpec((1, 1, RB), lambda i: (i, 0, 0)),
            pl.BlockSpec((EMB, 1), lambda i: (0, 0)),
            pl.BlockSpec((1, 1), lambda i: (0, 0)),
        ],
        out_specs=pl.BlockSpec((G, 1), lambda i: (0, 0)),
        out_shape=jax.ShapeDtypeStruct((G, 1), jnp.float32),
        scratch_shapes=[pltpu.VMEM((G, EMB), jnp.float32)],
    )(u, sums3, g, b, batch3, lin_W, lin_b)


def kernel(x, edge_index, edge_attr, batch, atom_tables, bond_tables, eps,
           mlp_W1, mlp_b1, mlp_bn_g, mlp_bn_b, mlp_W2, mlp_b2,
           bn_g, bn_b, lin_W, lin_b):
    f32 = jnp.float32

    xt = (jnp.zeros((9, NP), jnp.int32).at[:, :N].set(x.T.astype(jnp.int32))
          .reshape(9 * NP))
    atab = (atom_tables.astype(f32)
            .reshape(9, 128, NC, HALF).transpose(2, 0, 1, 3)
            .reshape(NC * 9 * 128, HALF))

    src = jnp.zeros((EPAD,), jnp.int32).at[:E].set(edge_index[0].astype(jnp.int32))
    dst = jnp.full((EPAD,), N, jnp.int32).at[:E].set(edge_index[1].astype(jnp.int32))
    ea = edge_attr.astype(jnp.int32)
    cidx = jnp.zeros((EPAD,), jnp.int32).at[:E].set(
        ea[:, 0] * 256 + ea[:, 1] * 16 + ea[:, 2])

    batch3 = (jnp.zeros((NP,), jnp.int32).at[:N].set(batch.astype(jnp.int32))
              .reshape(NBLK, 1, RB))

    ct = _build_bond_tables(bond_tables.astype(f32))
    eetabs = (ct.reshape(L, 4096, NC, HALF).transpose(0, 2, 1, 3)
              .reshape(L, NC * 4096, HALF))

    h = _atom_kernel(xt, atab)

    for l in range(L):
        agg = _msg_kernel(h, src, dst, cidx, eetabs[l])
        h0, h1 = h[:NP], h[NP:]
        a0, a1 = agg[:NP], agg[NP:]
        cmul = (1.0 + eps[l]).astype(f32).reshape(1, 1)
        w1 = mlp_W1[l].astype(f32)
        y, sums = _passA(h0, h1, a0, a1, w1[:HALF], w1[HALF:],
                         mlp_b1[l].reshape(1, EMB).astype(f32), cmul)
        u, sums2 = _passB(y, sums,
                          mlp_bn_g[l].reshape(1, EMB).astype(f32),
                          mlp_bn_b[l].reshape(1, EMB).astype(f32),
                          mlp_W2[l].astype(f32),
                          mlp_b2[l].reshape(1, EMB).astype(f32))
        if l < L - 1:
            h = _passC(u, sums2,
                       bn_g[l].reshape(1, EMB).astype(f32),
                       bn_b[l].reshape(1, EMB).astype(f32),
                       relu=True).reshape(NC * NP, HALF)
        else:
            out = _pool(u, sums2,
                        bn_g[l].reshape(1, EMB).astype(f32),
                        bn_b[l].reshape(1, EMB).astype(f32),
                        batch3, lin_W.astype(f32),
                        lin_b.reshape(1, 1).astype(f32))
    return out
